# Initial kernel scaffold; baseline (speedup 1.0000x reference)
import functools

import jax
import jax.numpy as jnp
from jax.experimental import pallas as pl
from jax.experimental.pallas import tpu as pltpu

_F32 = jnp.float32
_NEG_INF = float("-inf")


def _mlp3max_body(x_ref, w1, b1, w2, b2, w3, b3, out_ref):
    i = pl.program_id(0)
    h = x_ref[...]
    h = jnp.maximum(jnp.dot(h, w1[...], preferred_element_type=_F32) + b1[...], 0.0)
    h = jnp.maximum(jnp.dot(h, w2[...], preferred_element_type=_F32) + b2[...], 0.0)
    h = jnp.dot(h, w3[...], preferred_element_type=_F32) + b3[...]
    bm = jnp.max(h, axis=0, keepdims=True)

    @pl.when(i == 0)
    def _():
        out_ref[...] = bm

    @pl.when(i > 0)
    def _():
        out_ref[...] = jnp.maximum(out_ref[...], bm)


def _mlp3_max(xrows, mlp, block_rows):
    (w1, b1), (w2, b2), (w3, b3) = mlp
    n = xrows.shape[0]
    grid = n // block_rows
    full = lambda a: pl.BlockSpec(a.shape, lambda i: (0,) * a.ndim)
    args = (xrows, w1, b1.reshape(1, -1), w2, b2.reshape(1, -1), w3,
            b3.reshape(1, -1))
    return pl.pallas_call(
        _mlp3max_body,
        grid=(grid,),
        in_specs=[pl.BlockSpec((block_rows, xrows.shape[1]), lambda i: (i, 0))]
        + [full(a) for a in args[1:]],
        out_specs=pl.BlockSpec((1, w3.shape[1]), lambda i: (0, 0)),
        out_shape=jax.ShapeDtypeStruct((1, w3.shape[1]), _F32),
    )(*args)


def _mlp3row_body(x_ref, w1, b1, w2, b2, w3, b3, out_ref):
    h = x_ref[...]
    h = jnp.maximum(jnp.dot(h, w1[...], preferred_element_type=_F32) + b1[...], 0.0)
    h = jnp.maximum(jnp.dot(h, w2[...], preferred_element_type=_F32) + b2[...], 0.0)
    out_ref[...] = jnp.dot(h, w3[...], preferred_element_type=_F32) + b3[...]


def _mlp3_row(xrow, mlp):
    (w1, b1), (w2, b2), (w3, b3) = mlp
    args = (xrow, w1, b1.reshape(1, -1), w2, b2.reshape(1, -1), w3,
            b3.reshape(1, -1))
    return pl.pallas_call(
        _mlp3row_body,
        out_shape=jax.ShapeDtypeStruct((1, w3.shape[1]), _F32),
    )(*args)


def _transform_body(t_ref, pos_ref, out_ref):
    px, py, pz = pos_ref[0], pos_ref[1], pos_ref[2]
    for j in range(3):
        m0 = t_ref[0, j] + (1.0 if j == 0 else 0.0)
        m1 = t_ref[0, 3 + j] + (1.0 if j == 1 else 0.0)
        m2 = t_ref[0, 6 + j] + (1.0 if j == 2 else 0.0)
        out_ref[j] = (px * m0 + py * m1) + pz * m2


def _transform(t, pos_t):
    return pl.pallas_call(
        _transform_body,
        out_shape=jax.ShapeDtypeStruct(pos_t.shape, _F32),
    )(t, pos_t)


def _fps_body(pos_ref, out_ref, *, n_sample, n_pts):
    sub = n_pts // 128
    px, py, pz = pos_ref[0], pos_ref[1], pos_ref[2]
    row = jax.lax.broadcasted_iota(jnp.int32, (sub, 128), 0)
    col = jax.lax.broadcasted_iota(jnp.int32, (sub, 128), 1)
    flat = row * 128 + col
    qcol = jax.lax.broadcasted_iota(jnp.int32, (1, n_sample), 1)

    x0, y0, z0 = px[0, 0], py[0, 0], pz[0, 0]
    dx, dy, dz = px - x0, py - y0, pz - z0
    mind = (dx * dx + dy * dy) + dz * dz
    first = qcol == 0
    qx = jnp.where(first, x0, 0.0)
    qy = jnp.where(first, y0, 0.0)
    qz = jnp.where(first, z0, 0.0)

    def body(i, carry):
        mind, qx, qy, qz = carry
        m = jnp.max(mind)
        nxt = jnp.min(jnp.where(mind == m, flat, n_pts))
        sel = flat == nxt
        bx = jnp.sum(jnp.where(sel, px, 0.0))
        by = jnp.sum(jnp.where(sel, py, 0.0))
        bz = jnp.sum(jnp.where(sel, pz, 0.0))
        dx, dy, dz = px - bx, py - by, pz - bz
        d = (dx * dx + dy * dy) + dz * dz
        mind = jnp.minimum(mind, d)
        hit = qcol == i
        qx = jnp.where(hit, bx, qx)
        qy = jnp.where(hit, by, qy)
        qz = jnp.where(hit, bz, qz)
        return mind, qx, qy, qz

    _, qx, qy, qz = jax.lax.fori_loop(1, n_sample, body, (mind, qx, qy, qz))
    out_ref[0:1, :] = qx
    out_ref[1:2, :] = qy
    out_ref[2:3, :] = qz


def _fps(pos_t3, n_sample):
    n_pts = pos_t3.shape[1] * 128
    return pl.pallas_call(
        functools.partial(_fps_body, n_sample=n_sample, n_pts=n_pts),
        out_shape=jax.ShapeDtypeStruct((3, n_sample), _F32),
    )(pos_t3)


def _group_body(posq_ref, src_ref, inds_ref, vals_ref, d_ref, *, n_src, k, qb):
    qx = posq_ref[:, 0:1]
    qy = posq_ref[:, 1:2]
    qz = posq_ref[:, 2:3]
    sx = src_ref[0:1, :]
    sy = src_ref[1:2, :]
    sz = src_ref[2:3, :]
    dx, dy, dz = qx - sx, qy - sy, qz - sz
    d_ref[...] = (dx * dx + dy * dy) + dz * dz
    kcol = jax.lax.broadcasted_iota(jnp.int32, (qb, k), 1)

    def body(j, carry):
        inds, vals = carry
        dm = d_ref[...]
        i_n = jax.lax.broadcasted_iota(jnp.int32, (qb, n_src), 1)
        v = jnp.min(dm, axis=1, keepdims=True)
        nxt = jnp.min(jnp.where(dm == v, i_n, n_src), axis=1, keepdims=True)
        d_ref[...] = jnp.where(i_n == nxt, jnp.inf, dm)
        hit = kcol == j
        inds = jnp.where(hit, nxt, inds)
        vals = jnp.where(hit, v, vals)
        return inds, vals

    inds, vals = jax.lax.fori_loop(
        0, k, body,
        (jnp.zeros((qb, k), jnp.int32), jnp.zeros((qb, k), _F32)))
    inds_ref[...] = inds
    vals_ref[...] = vals


def _group(posq, src_t, k, qb):
    m = posq.shape[0]
    n_src = src_t.shape[1]
    return pl.pallas_call(
        functools.partial(_group_body, n_src=n_src, k=k, qb=qb),
        grid=(m // qb,),
        in_specs=[pl.BlockSpec((qb, 3), lambda i: (i, 0)),
                  pl.BlockSpec((3, n_src), lambda i: (0, 0))],
        out_specs=[pl.BlockSpec((qb, k), lambda i: (i, 0)),
                   pl.BlockSpec((qb, k), lambda i: (i, 0))],
        out_shape=[jax.ShapeDtypeStruct((m, k), jnp.int32),
                   jax.ShapeDtypeStruct((m, k), _F32)],
        scratch_shapes=[pltpu.VMEM((qb, n_src), _F32)],
    )(posq, src_t)


def _conv_body(g_ref, posq_ref, d2k_ref, wr, w2, b2, w3, b3, out_ref,
               *, qb, k, r2):
    c1 = wr.shape[1]
    c3 = w3.shape[1]
    lq = jnp.dot(posq_ref[...], wr[...], preferred_element_type=_F32)
    h = g_ref[...].reshape(qb, k, c1) - lq[:, None, :]
    h = jnp.maximum(h, 0.0).reshape(qb * k, c1)
    h = jnp.maximum(jnp.dot(h, w2[...], preferred_element_type=_F32) + b2[...], 0.0)
    h = jnp.dot(h, w3[...], preferred_element_type=_F32) + b3[...]
    valid = d2k_ref[...] <= r2
    msg = jnp.where(valid[:, :, None], h.reshape(qb, k, c3), _NEG_INF)
    out_ref[...] = jnp.max(msg, axis=1)


def _conv(gath, posq, d2k, wr, w2, b2, w3, b3, r2, qb):
    m, k = d2k.shape
    c1 = wr.shape[1]
    c3 = w3.shape[1]
    full = lambda a: pl.BlockSpec(a.shape, lambda i: (0,) * a.ndim)
    args = (gath, posq, d2k, wr, w2, b2.reshape(1, -1), w3, b3.reshape(1, -1))
    return pl.pallas_call(
        functools.partial(_conv_body, qb=qb, k=k, r2=r2),
        grid=(m // qb,),
        in_specs=[pl.BlockSpec((qb * k, c1), lambda i: (i, 0)),
                  pl.BlockSpec((qb, 3), lambda i: (i, 0)),
                  pl.BlockSpec((qb, k), lambda i: (i, 0))]
        + [full(a) for a in args[3:]],
        out_specs=pl.BlockSpec((qb, c3), lambda i: (i, 0)),
        out_shape=jax.ShapeDtypeStruct((m, c3), _F32),
    )(*args)


def _linear_body(x_ref, w, b, out_ref):
    out_ref[...] = jnp.dot(x_ref[...], w[...], preferred_element_type=_F32) + b[...]


def _linear(xrows, w, b, block_rows):
    n = xrows.shape[0]
    full = lambda a: pl.BlockSpec(a.shape, lambda i: (0,) * a.ndim)
    args = (xrows, w, b.reshape(1, -1))
    return pl.pallas_call(
        _linear_body,
        grid=(n // block_rows,),
        in_specs=[pl.BlockSpec((block_rows, xrows.shape[1]), lambda i: (i, 0)),
                  full(w), full(args[2])],
        out_specs=pl.BlockSpec((block_rows, w.shape[1]), lambda i: (i, 0)),
        out_shape=jax.ShapeDtypeStruct((n, w.shape[1]), _F32),
    )(*args)


def _stage(x_feat, pos_src, posq, d2k, nidx, mlps, radii, qb):
    flat_idx = nidx.reshape(-1)
    xin = jnp.concatenate([x_feat, pos_src], axis=1)
    nf = x_feat.shape[1]
    feats = []
    for r, mlp in zip(radii, mlps):
        (w1, b1), (w2, b2), (w3, b3) = mlp
        table = _linear(xin, w1, b1, block_rows=min(1024, xin.shape[0]))
        gath = table[flat_idx]
        feats.append(_conv(gath, posq, d2k, w1[nf:], w2, b2, w3, b3,
                           r * r, qb))
    return jnp.concatenate(feats, axis=1)


def kernel(pos, x, batch, params):
    n = pos.shape[0]
    p = params

    g = _mlp3_max(jnp.concatenate([pos, x], axis=1), p["tnet1"], 1024)
    t = _mlp3_row(g, p["tnet2"])
    pos_t = pos.T.reshape(3, n // 128, 128)
    tpos_t3 = _transform(t, pos_t)
    tpos_t = tpos_t3.reshape(3, n)
    tpos = tpos_t.T

    q1_t = _fps(tpos_t3, n // 4)
    posq1 = q1_t.T
    nidx1, d2k1 = _group(posq1, tpos_t, 64, 128)
    x1 = _stage(x, tpos, posq1, d2k1, nidx1, p["sa1"], (1.0, 2.0, 4.0), 128)

    q2_t = _fps(q1_t.reshape(3, (n // 4) // 128, 128), n // 8)
    posq2 = q2_t.T
    nidx2, d2k2 = _group(posq2, q1_t, 64, 128)
    x2 = _stage(x1, posq1, posq2, d2k2, nidx2, p["sa2"], (2.0, 4.0, 8.0), 64)

    g3 = _mlp3_max(jnp.concatenate([x2, posq2], axis=1), p["sa3"], 1024)
    return _mlp3_row(g3, p["head"])


# speedup vs baseline: 7.0767x; 7.0767x over previous
import functools

import jax
import jax.numpy as jnp
from jax.experimental import pallas as pl
from jax.experimental.pallas import tpu as pltpu

_F32 = jnp.float32
_NEG_INF = float("-inf")


def _mlp3max_body(x_ref, w1, b1, w2, b2, w3, b3, out_ref):
    i = pl.program_id(0)
    h = x_ref[...]
    h = jnp.maximum(jnp.dot(h, w1[...], preferred_element_type=_F32) + b1[...], 0.0)
    h = jnp.maximum(jnp.dot(h, w2[...], preferred_element_type=_F32) + b2[...], 0.0)
    h = jnp.dot(h, w3[...], preferred_element_type=_F32) + b3[...]
    bm = jnp.max(h, axis=0, keepdims=True)

    @pl.when(i == 0)
    def _():
        out_ref[...] = bm

    @pl.when(i > 0)
    def _():
        out_ref[...] = jnp.maximum(out_ref[...], bm)


def _mlp3_max(xrows, mlp, block_rows):
    (w1, b1), (w2, b2), (w3, b3) = mlp
    n = xrows.shape[0]
    grid = n // block_rows
    full = lambda a: pl.BlockSpec(a.shape, lambda i: (0,) * a.ndim)
    args = (xrows, w1, b1.reshape(1, -1), w2, b2.reshape(1, -1), w3,
            b3.reshape(1, -1))
    return pl.pallas_call(
        _mlp3max_body,
        grid=(grid,),
        in_specs=[pl.BlockSpec((block_rows, xrows.shape[1]), lambda i: (i, 0))]
        + [full(a) for a in args[1:]],
        out_specs=pl.BlockSpec((1, w3.shape[1]), lambda i: (0, 0)),
        out_shape=jax.ShapeDtypeStruct((1, w3.shape[1]), _F32),
    )(*args)


def _row_mlp(xrow, mlp):
    h = xrow
    for li, (w, b) in enumerate(mlp):
        h = h @ w + b
        if li < len(mlp) - 1:
            h = jax.nn.relu(h)
    return h


def _transform_body(t_ref, pos_ref, out_ref):
    px, py, pz = pos_ref[0], pos_ref[1], pos_ref[2]
    for j in range(3):
        m0 = t_ref[0, j] + (1.0 if j == 0 else 0.0)
        m1 = t_ref[0, 3 + j] + (1.0 if j == 1 else 0.0)
        m2 = t_ref[0, 6 + j] + (1.0 if j == 2 else 0.0)
        out_ref[j] = (px * m0 + py * m1) + pz * m2


def _transform(t, pos_t):
    return pl.pallas_call(
        _transform_body,
        out_shape=jax.ShapeDtypeStruct(pos_t.shape, _F32),
    )(t, pos_t)


def _fps_body(pos_ref, out_ref, *, n_sample, n_pts):
    sub = n_pts // 128
    px, py, pz = pos_ref[0], pos_ref[1], pos_ref[2]
    row = jax.lax.broadcasted_iota(jnp.int32, (sub, 128), 0)
    col = jax.lax.broadcasted_iota(jnp.int32, (sub, 128), 1)
    flat = row * 128 + col
    qcol = jax.lax.broadcasted_iota(jnp.int32, (1, n_sample), 1)

    x0, y0, z0 = px[0, 0], py[0, 0], pz[0, 0]
    dx, dy, dz = px - x0, py - y0, pz - z0
    mind = (dx * dx + dy * dy) + dz * dz
    first = qcol == 0
    qx = jnp.where(first, x0, 0.0)
    qy = jnp.where(first, y0, 0.0)
    qz = jnp.where(first, z0, 0.0)

    def body(i, carry):
        mind, qx, qy, qz = carry
        m = jnp.max(mind)
        nxt = jnp.min(jnp.where(mind == m, flat, n_pts))
        sel = flat == nxt
        bx = jnp.sum(jnp.where(sel, px, 0.0))
        by = jnp.sum(jnp.where(sel, py, 0.0))
        bz = jnp.sum(jnp.where(sel, pz, 0.0))
        dx, dy, dz = px - bx, py - by, pz - bz
        d = (dx * dx + dy * dy) + dz * dz
        mind = jnp.minimum(mind, d)
        hit = qcol == i
        qx = jnp.where(hit, bx, qx)
        qy = jnp.where(hit, by, qy)
        qz = jnp.where(hit, bz, qz)
        return mind, qx, qy, qz

    _, qx, qy, qz = jax.lax.fori_loop(1, n_sample, body, (mind, qx, qy, qz))
    out_ref[0:1, :] = qx
    out_ref[1:2, :] = qy
    out_ref[2:3, :] = qz


def _fps(pos_t3, n_sample):
    n_pts = pos_t3.shape[1] * 128
    return pl.pallas_call(
        functools.partial(_fps_body, n_sample=n_sample, n_pts=n_pts),
        out_shape=jax.ShapeDtypeStruct((3, n_sample), _F32),
    )(pos_t3)


def _group_body(posq_ref, src_ref, inds_ref, vals_ref, d_ref, *, n_src, k, qb):
    qx = posq_ref[:, 0:1]
    qy = posq_ref[:, 1:2]
    qz = posq_ref[:, 2:3]
    sx = src_ref[0:1, :]
    sy = src_ref[1:2, :]
    sz = src_ref[2:3, :]
    dx, dy, dz = qx - sx, qy - sy, qz - sz
    d_ref[...] = (dx * dx + dy * dy) + dz * dz
    kcol = jax.lax.broadcasted_iota(jnp.int32, (qb, k), 1)

    def body(j, carry):
        inds, vals = carry
        dm = d_ref[...]
        i_n = jax.lax.broadcasted_iota(jnp.int32, (qb, n_src), 1)
        v = jnp.min(dm, axis=1, keepdims=True)
        nxt = jnp.min(jnp.where(dm == v, i_n, n_src), axis=1, keepdims=True)
        d_ref[...] = jnp.where(i_n == nxt, jnp.inf, dm)
        hit = kcol == j
        inds = jnp.where(hit, nxt, inds)
        vals = jnp.where(hit, v, vals)
        return inds, vals

    inds, vals = jax.lax.fori_loop(
        0, k, body,
        (jnp.zeros((qb, k), jnp.int32), jnp.zeros((qb, k), _F32)))
    inds_ref[...] = inds
    vals_ref[...] = vals


def _group(posq, src_t, k, qb):
    m = posq.shape[0]
    n_src = src_t.shape[1]
    return pl.pallas_call(
        functools.partial(_group_body, n_src=n_src, k=k, qb=qb),
        grid=(m // qb,),
        in_specs=[pl.BlockSpec((qb, 3), lambda i: (i, 0)),
                  pl.BlockSpec((3, n_src), lambda i: (0, 0))],
        out_specs=[pl.BlockSpec((qb, k), lambda i: (i, 0)),
                   pl.BlockSpec((qb, k), lambda i: (i, 0))],
        out_shape=[jax.ShapeDtypeStruct((m, k), jnp.int32),
                   jax.ShapeDtypeStruct((m, k), _F32)],
        scratch_shapes=[pltpu.VMEM((qb, n_src), _F32)],
    )(posq, src_t)


def _conv_body(g_ref, pq_ref, d2f_ref, w1, b1, w2, b2, w3, b3, out_ref,
               *, qb, k, r2):
    c3 = w3.shape[1]
    h = g_ref[...] - pq_ref[...]
    h = jnp.maximum(jnp.dot(h, w1[...], preferred_element_type=_F32) + b1[...], 0.0)
    h = jnp.maximum(jnp.dot(h, w2[...], preferred_element_type=_F32) + b2[...], 0.0)
    h = jnp.dot(h, w3[...], preferred_element_type=_F32) + b3[...]
    valid = d2f_ref[...] <= r2
    h = jnp.where(valid, h, _NEG_INF)
    out_ref[...] = jnp.max(h.reshape(qb, k, c3), axis=1)


def _conv(gath, pqpad, d2flat, w1, b1, w2, b2, w3, b3, r2, qb, nk):
    mk, cin = gath.shape
    c3 = w3.shape[1]
    full = lambda a: pl.BlockSpec(a.shape, lambda i: (0,) * a.ndim)
    args = (gath, pqpad, d2flat, w1, b1.reshape(1, -1), w2,
            b2.reshape(1, -1), w3, b3.reshape(1, -1))
    m = mk // nk
    return pl.pallas_call(
        functools.partial(_conv_body, qb=qb, k=nk, r2=r2),
        grid=(m // qb,),
        in_specs=[pl.BlockSpec((qb * nk, cin), lambda i: (i, 0)),
                  pl.BlockSpec((qb * nk, cin), lambda i: (i, 0)),
                  pl.BlockSpec((qb * nk, 1), lambda i: (i, 0))]
        + [full(a) for a in args[3:]],
        out_specs=pl.BlockSpec((qb, c3), lambda i: (i, 0)),
        out_shape=jax.ShapeDtypeStruct((m, c3), _F32),
    )(*args)


def _stage(x_feat, pos_src, posq, d2k, nidx, mlps, radii, qb):
    k = nidx.shape[1]
    mk = nidx.size
    flat_idx = nidx.reshape(-1)
    xin = jnp.concatenate([x_feat, pos_src], axis=1)
    nf = x_feat.shape[1]
    gath = xin[flat_idx]
    pqpad = jnp.concatenate(
        [jnp.zeros((mk, nf), _F32), jnp.repeat(posq, k, axis=0)], axis=1)
    d2flat = d2k.reshape(-1, 1)
    feats = []
    for r, mlp in zip(radii, mlps):
        (w1, b1), (w2, b2), (w3, b3) = mlp
        feats.append(_conv(gath, pqpad, d2flat, w1, b1, w2, b2, w3, b3,
                           r * r, qb, k))
    return jnp.concatenate(feats, axis=1)


def kernel(pos, x, batch, params):
    n = pos.shape[0]
    p = params

    g = _mlp3_max(jnp.concatenate([pos, x], axis=1), p["tnet1"], 1024)
    t = _row_mlp(g, p["tnet2"])
    pos_t = pos.T.reshape(3, n // 128, 128)
    tpos_t3 = _transform(t, pos_t)
    tpos_t = tpos_t3.reshape(3, n)
    tpos = tpos_t.T

    q1_t = _fps(tpos_t3, n // 4)
    posq1 = q1_t.T
    nidx1, d2k1 = _group(posq1, tpos_t, 64, 128)
    x1 = _stage(x, tpos, posq1, d2k1, nidx1, p["sa1"], (1.0, 2.0, 4.0), 128)

    q2_t = _fps(q1_t.reshape(3, (n // 4) // 128, 128), n // 8)
    posq2 = q2_t.T
    nidx2, d2k2 = _group(posq2, q1_t, 64, 128)
    x2 = _stage(x1, posq1, posq2, d2k2, nidx2, p["sa2"], (2.0, 4.0, 8.0), 64)

    g3 = _mlp3_max(jnp.concatenate([x2, posq2], axis=1), p["sa3"], 1024)
    return _row_mlp(g3, p["head"])


# speedup vs baseline: 8.8185x; 1.2461x over previous
import functools

import jax
import jax.numpy as jnp
from jax import lax
from jax.experimental import pallas as pl
from jax.experimental.pallas import tpu as pltpu
from jax.experimental.pallas import tpu_sc as plsc

_F32 = jnp.float32
_NEG_INF = float("-inf")


def _mlp3max_body(x_ref, w1, b1, w2, b2, w3, b3, out_ref):
    i = pl.program_id(0)
    h = x_ref[...]
    h = jnp.maximum(jnp.dot(h, w1[...], preferred_element_type=_F32) + b1[...], 0.0)
    h = jnp.maximum(jnp.dot(h, w2[...], preferred_element_type=_F32) + b2[...], 0.0)
    h = jnp.dot(h, w3[...], preferred_element_type=_F32) + b3[...]
    bm = jnp.max(h, axis=0, keepdims=True)

    @pl.when(i == 0)
    def _():
        out_ref[...] = bm

    @pl.when(i > 0)
    def _():
        out_ref[...] = jnp.maximum(out_ref[...], bm)


def _mlp3_max(xrows, mlp, block_rows):
    (w1, b1), (w2, b2), (w3, b3) = mlp
    n = xrows.shape[0]
    grid = n // block_rows
    full = lambda a: pl.BlockSpec(a.shape, lambda i: (0,) * a.ndim)
    args = (xrows, w1, b1.reshape(1, -1), w2, b2.reshape(1, -1), w3,
            b3.reshape(1, -1))
    return pl.pallas_call(
        _mlp3max_body,
        grid=(grid,),
        in_specs=[pl.BlockSpec((block_rows, xrows.shape[1]), lambda i: (i, 0))]
        + [full(a) for a in args[1:]],
        out_specs=pl.BlockSpec((1, w3.shape[1]), lambda i: (0, 0)),
        out_shape=jax.ShapeDtypeStruct((1, w3.shape[1]), _F32),
    )(*args)


def _row_mlp(xrow, mlp):
    h = xrow
    for li, (w, b) in enumerate(mlp):
        h = h @ w + b
        if li < len(mlp) - 1:
            h = jax.nn.relu(h)
    return h


def _transform_body(t_ref, pos_ref, out_ref):
    px, py, pz = pos_ref[0], pos_ref[1], pos_ref[2]
    for j in range(3):
        m0 = t_ref[0, j] + (1.0 if j == 0 else 0.0)
        m1 = t_ref[0, 3 + j] + (1.0 if j == 1 else 0.0)
        m2 = t_ref[0, 6 + j] + (1.0 if j == 2 else 0.0)
        out_ref[j] = (px * m0 + py * m1) + pz * m2


def _transform(t, pos_t):
    return pl.pallas_call(
        _transform_body,
        out_shape=jax.ShapeDtypeStruct(pos_t.shape, _F32),
    )(t, pos_t)


def _fps_body(pos_ref, out_ref, *, n_sample, n_pts):
    sub = n_pts // 128
    px, py, pz = pos_ref[0], pos_ref[1], pos_ref[2]
    row = jax.lax.broadcasted_iota(jnp.int32, (sub, 128), 0)
    col = jax.lax.broadcasted_iota(jnp.int32, (sub, 128), 1)
    flat = row * 128 + col
    qcol = jax.lax.broadcasted_iota(jnp.int32, (1, n_sample), 1)

    x0, y0, z0 = px[0, 0], py[0, 0], pz[0, 0]
    dx, dy, dz = px - x0, py - y0, pz - z0
    mind = (dx * dx + dy * dy) + dz * dz
    first = qcol == 0
    qx = jnp.where(first, x0, 0.0)
    qy = jnp.where(first, y0, 0.0)
    qz = jnp.where(first, z0, 0.0)

    def body(i, carry):
        mind, qx, qy, qz = carry
        m = jnp.max(mind)
        nxt = jnp.min(jnp.where(mind == m, flat, n_pts))
        sel = flat == nxt
        bx = jnp.sum(jnp.where(sel, px, 0.0))
        by = jnp.sum(jnp.where(sel, py, 0.0))
        bz = jnp.sum(jnp.where(sel, pz, 0.0))
        dx, dy, dz = px - bx, py - by, pz - bz
        d = (dx * dx + dy * dy) + dz * dz
        mind = jnp.minimum(mind, d)
        hit = qcol == i
        qx = jnp.where(hit, bx, qx)
        qy = jnp.where(hit, by, qy)
        qz = jnp.where(hit, bz, qz)
        return mind, qx, qy, qz

    _, qx, qy, qz = jax.lax.fori_loop(1, n_sample, body, (mind, qx, qy, qz))
    out_ref[0:1, :] = qx
    out_ref[1:2, :] = qy
    out_ref[2:3, :] = qz


def _fps(pos_t3, n_sample):
    n_pts = pos_t3.shape[1] * 128
    return pl.pallas_call(
        functools.partial(_fps_body, n_sample=n_sample, n_pts=n_pts),
        out_shape=jax.ShapeDtypeStruct((3, n_sample), _F32),
    )(pos_t3)


def _group_body(posq_ref, src_ref, inds_ref, vals_ref, d_ref, *, n_src, k, qb):
    qx = posq_ref[:, 0:1]
    qy = posq_ref[:, 1:2]
    qz = posq_ref[:, 2:3]
    sx = src_ref[0:1, :]
    sy = src_ref[1:2, :]
    sz = src_ref[2:3, :]
    dx, dy, dz = qx - sx, qy - sy, qz - sz
    d_ref[...] = (dx * dx + dy * dy) + dz * dz
    kcol = jax.lax.broadcasted_iota(jnp.int32, (qb, k), 1)

    def body(j, carry):
        inds, vals = carry
        dm = d_ref[...]
        i_n = jax.lax.broadcasted_iota(jnp.int32, (qb, n_src), 1)
        v = jnp.min(dm, axis=1, keepdims=True)
        nxt = jnp.min(jnp.where(dm == v, i_n, n_src), axis=1, keepdims=True)
        d_ref[...] = jnp.where(i_n == nxt, jnp.inf, dm)
        hit = kcol == j
        inds = jnp.where(hit, nxt, inds)
        vals = jnp.where(hit, v, vals)
        return inds, vals

    inds, vals = jax.lax.fori_loop(
        0, k, body,
        (jnp.zeros((qb, k), jnp.int32), jnp.zeros((qb, k), _F32)))
    inds_ref[...] = inds
    vals_ref[...] = vals


def _group(posq, src_t, k, qb):
    m = posq.shape[0]
    n_src = src_t.shape[1]
    return pl.pallas_call(
        functools.partial(_group_body, n_src=n_src, k=k, qb=qb),
        grid=(m // qb,),
        in_specs=[pl.BlockSpec((qb, 3), lambda i: (i, 0)),
                  pl.BlockSpec((3, n_src), lambda i: (0, 0))],
        out_specs=[pl.BlockSpec((qb, k), lambda i: (i, 0)),
                   pl.BlockSpec((qb, k), lambda i: (i, 0))],
        out_shape=[jax.ShapeDtypeStruct((m, k), jnp.int32),
                   jax.ShapeDtypeStruct((m, k), _F32)],
        scratch_shapes=[pltpu.VMEM((qb, n_src), _F32)],
    )(posq, src_t)


def _conv3_body(g_ref, pq_ref, d2f_ref, *rest, qb, k, r2s):
    wrefs, outs = rest[:18], rest[18:]
    inp = g_ref[...] - pq_ref[...]
    for c in range(3):
        w1, b1, w2, b2, w3, b3 = wrefs[6 * c:6 * c + 6]
        c3 = w3.shape[1]
        h = jnp.maximum(
            jnp.dot(inp, w1[...], preferred_element_type=_F32) + b1[...], 0.0)
        h = jnp.maximum(
            jnp.dot(h, w2[...], preferred_element_type=_F32) + b2[...], 0.0)
        h = jnp.dot(h, w3[...], preferred_element_type=_F32) + b3[...]
        valid = d2f_ref[...] <= r2s[c]
        h = jnp.where(valid, h, _NEG_INF)
        outs[c][...] = jnp.max(h.reshape(qb, k, c3), axis=1)


def _conv3(gath, pqpad, d2flat, mlps, r2s, qb, nk):
    mk, cin = gath.shape
    m = mk // nk
    full = lambda a: pl.BlockSpec(a.shape, lambda i: (0,) * a.ndim)
    wargs = []
    for (w1, b1), (w2, b2), (w3, b3) in mlps:
        wargs += [w1, b1.reshape(1, -1), w2, b2.reshape(1, -1), w3,
                  b3.reshape(1, -1)]
    c3s = [mlp[2][0].shape[1] for mlp in mlps]
    return pl.pallas_call(
        functools.partial(_conv3_body, qb=qb, k=nk, r2s=tuple(r2s)),
        grid=(m // qb,),
        in_specs=[pl.BlockSpec((qb * nk, cin), lambda i: (i, 0)),
                  pl.BlockSpec((qb * nk, cin), lambda i: (i, 0)),
                  pl.BlockSpec((qb * nk, 1), lambda i: (i, 0))]
        + [full(a) for a in wargs],
        out_specs=[pl.BlockSpec((qb, c3), lambda i: (i, 0)) for c3 in c3s],
        out_shape=[jax.ShapeDtypeStruct((m, c3), _F32) for c3 in c3s],
    )(gath, pqpad, d2flat, *wargs)


def _sc_gather(table, idx):
    v_rows, d = table.shape
    b = idx.shape[0]
    nw = 32
    bpw = b // nw
    ch = min(bpw, 128)
    nch = bpw // ch
    mesh = plsc.VectorSubcoreMesh(core_axis_name="c", subcore_axis_name="s")

    @functools.partial(
        pl.kernel,
        mesh=mesh,
        out_type=jax.ShapeDtypeStruct((b, d), _F32),
        scratch_types=[
            pltpu.VMEM((ch,), jnp.int32),
            pltpu.VMEM((ch, d), _F32),
            pltpu.SemaphoreType.DMA,
        ],
    )
    def k(table_hbm, idx_hbm, out_hbm, idx_v, rows_v, sem):
        wid = lax.axis_index("s") * 2 + lax.axis_index("c")
        base = wid * bpw

        def body(ci, carry):
            off = base + ci * ch
            pltpu.sync_copy(idx_hbm.at[pl.ds(off, ch)], idx_v)
            pltpu.async_copy(table_hbm.at[idx_v], rows_v, sem).wait()
            pltpu.sync_copy(rows_v, out_hbm.at[pl.ds(off, ch)])
            return carry

        lax.fori_loop(0, nch, body, 0)

    return k(table, idx)


def _pad_cols(a, d):
    return jnp.pad(a, ((0, 0), (0, d - a.shape[1])))


def _stage(x_feat, pos_src, posq, d2k, nidx, mlps, radii, qb):
    k = nidx.shape[1]
    mk = nidx.size
    flat_idx = nidx.reshape(-1)
    xin = jnp.concatenate([x_feat, pos_src], axis=1)
    nf = x_feat.shape[1]
    cin = xin.shape[1]
    dpad = -(-cin // 128) * 128
    gath = _sc_gather(_pad_cols(xin, dpad), flat_idx)
    pqpad = jnp.concatenate(
        [jnp.zeros((mk, nf), _F32), jnp.repeat(posq, k, axis=0),
         jnp.zeros((mk, dpad - cin), _F32)], axis=1)
    d2flat = d2k.reshape(-1, 1)
    mlps_p = [((jnp.pad(w1, ((0, dpad - cin), (0, 0))), b1), l2, l3)
              for (w1, b1), l2, l3 in mlps]
    feats = _conv3(gath, pqpad, d2flat, mlps_p, [r * r for r in radii],
                   qb, k)
    return jnp.concatenate(feats, axis=1)


def kernel(pos, x, batch, params):
    n = pos.shape[0]
    p = params

    g = _mlp3_max(jnp.concatenate([pos, x], axis=1), p["tnet1"], 1024)
    t = _row_mlp(g, p["tnet2"])
    pos_t = pos.T.reshape(3, n // 128, 128)
    tpos_t3 = _transform(t, pos_t)
    tpos_t = tpos_t3.reshape(3, n)
    tpos = tpos_t.T

    q1_t = _fps(tpos_t3, n // 4)
    posq1 = q1_t.T
    nidx1, d2k1 = _group(posq1, tpos_t, 64, 128)
    x1 = _stage(x, tpos, posq1, d2k1, nidx1, p["sa1"], (1.0, 2.0, 4.0), 128)

    q2_t = _fps(q1_t.reshape(3, (n // 4) // 128, 128), n // 8)
    posq2 = q2_t.T
    nidx2, d2k2 = _group(posq2, q1_t, 64, 128)
    x2 = _stage(x1, posq1, posq2, d2k2, nidx2, p["sa2"], (2.0, 4.0, 8.0), 64)

    g3 = _mlp3_max(jnp.concatenate([x2, posq2], axis=1), p["sa3"], 1024)
    return _row_mlp(g3, p["head"])
